# Initial kernel scaffold; baseline (speedup 1.0000x reference)
import jax
import jax.numpy as jnp
from jax.experimental import pallas as pl


def kernel(x, edge_index, edge_vals, kernel, bias):
    raise NotImplementedError("write your pallas kernel here")


# speedup vs baseline: 2.6986x; 2.6986x over previous
import functools

import jax
import jax.numpy as jnp
from jax import lax
from jax.experimental import pallas as pl
from jax.experimental.pallas import tpu as pltpu
from jax.experimental.pallas import tpu_sc as plsc

M = 10000
FIN = 128
FOUT = 128
KPOLY = 4
E = 320000
NB = 2

NT = 16
EPT = E // NT
CH = 80
NCHUNK = EPT // CH
RCH = 40
NRC = M // RCH
RR = -(-NRC // NT)

_LANES = 16
_FSL = FIN // _LANES


def _make_spmm(alpha, beta, gamma):
    mesh = plsc.VectorSubcoreMesh(core_axis_name="c", subcore_axis_name="s")

    @functools.partial(
        pl.kernel,
        mesh=mesh,
        out_type=jax.ShapeDtypeStruct((NB * M, FIN), jnp.float32),
        scratch_types=[
            pltpu.VMEM_SHARED((M, FIN), jnp.float32),
            pltpu.VMEM((CH, FIN), jnp.float32),
            pltpu.VMEM((CH,), jnp.int32),
            pltpu.VMEM((CH,), jnp.int32),
            pltpu.VMEM((CH,), jnp.float32),
            pltpu.VMEM((RCH, FIN), jnp.float32),
            pltpu.VMEM((RCH, FIN), jnp.float32),
            pltpu.VMEM((RCH, FIN), jnp.float32),
            pltpu.SemaphoreType.DMA,
        ],
    )
    def spmm(z_hbm, w_hbm, row_hbm, col_hbm, val_hbm, out_hbm,
             acc_sh, rows_v, colidx_v, rowidx_v, val_v, zb, vb, wb, sem):
        c = lax.axis_index("c")
        s = lax.axis_index("s")
        cM = c * M

        zvec = jnp.zeros((_LANES,), jnp.float32)

        def zero_body(r, _):
            for j in range(_FSL):
                zb[r, pl.ds(_LANES * j, _LANES)] = zvec
            return 0

        lax.fori_loop(0, RCH, zero_body, 0)

        def zero_chunk(i, _):
            j = s + NT * i

            @pl.when(j < NRC)
            def _():
                r0 = pl.multiple_of(j * RCH, 8)
                pltpu.sync_copy(zb, acc_sh.at[pl.ds(r0, RCH)])

            return 0

        lax.fori_loop(0, RR, zero_chunk, 0)
        plsc.subcore_barrier()

        def chunk_body(k, _):
            e0 = s * EPT + k * CH
            pltpu.sync_copy(row_hbm.at[pl.ds(e0, CH)], rowidx_v)
            pltpu.sync_copy(col_hbm.at[pl.ds(e0, CH)], colidx_v)
            pltpu.sync_copy(val_hbm.at[pl.ds(e0, CH)], val_v)
            off = jnp.full((_LANES,), cM, jnp.int32)
            for j in range(CH // _LANES):
                sl = pl.ds(_LANES * j, _LANES)
                colidx_v[sl] = colidx_v[sl] + off
            pltpu.async_copy(z_hbm.at[colidx_v], rows_v, sem).wait()

            def group_body(g, _):
                vals16 = val_v[pl.ds(_LANES * g, _LANES)]
                for l in range(_LANES):
                    sv = lax.gather(
                        vals16,
                        jnp.full((_LANES, 1), l, jnp.int32),
                        lax.GatherDimensionNumbers(
                            offset_dims=(), collapsed_slice_dims=(0,),
                            start_index_map=(0,)),
                        slice_sizes=(1,),
                        mode=lax.GatherScatterMode.PROMISE_IN_BOUNDS)
                    e = g * _LANES + l
                    for j in range(_FSL):
                        sl = pl.ds(_LANES * j, _LANES)
                        rows_v[e, sl] = rows_v[e, sl] * sv
                return 0

            lax.fori_loop(0, CH // _LANES, group_body, 0)
            pltpu.sync_copy(rows_v, acc_sh.at[rowidx_v], add=True)
            return 0

        lax.fori_loop(0, NCHUNK, chunk_body, 0)
        plsc.subcore_barrier()

        def epi_chunk(i, _):
            jc = s + NT * i

            @pl.when(jc < NRC)
            def _():
                r0 = pl.multiple_of(jc * RCH, 8)
                g0 = pl.multiple_of(cM + r0, 8)
                pltpu.sync_copy(acc_sh.at[pl.ds(r0, RCH)], zb)
                pltpu.sync_copy(z_hbm.at[pl.ds(g0, RCH)], vb)
                if gamma != 0.0:
                    pltpu.sync_copy(w_hbm.at[pl.ds(g0, RCH)], wb)

                def comb_body(r, _):
                    for j in range(_FSL):
                        sl = pl.ds(_LANES * j, _LANES)
                        res = alpha * zb[r, sl] + beta * vb[r, sl]
                        if gamma != 0.0:
                            res = res + gamma * wb[r, sl]
                        zb[r, sl] = res
                    return 0

                lax.fori_loop(0, RCH, comb_body, 0)
                pltpu.sync_copy(zb, out_hbm.at[pl.ds(g0, RCH)])

            return 0

        lax.fori_loop(0, RR, epi_chunk, 0)

    return spmm


_spmm_first = _make_spmm(1.0, -1.0, 0.0)
_spmm_rec = _make_spmm(2.0, -2.0, -1.0)


TROW = 2000
NBLK = M // TROW


def _tail_body(z0_ref, z1_ref, z2_ref, z3_ref, w_ref, b_ref, out_ref):
    acc = jnp.dot(z0_ref[...], w_ref[0], preferred_element_type=jnp.float32)
    acc += jnp.dot(z1_ref[...], w_ref[1], preferred_element_type=jnp.float32)
    acc += jnp.dot(z2_ref[...], w_ref[2], preferred_element_type=jnp.float32)
    acc += jnp.dot(z3_ref[...], w_ref[3], preferred_element_type=jnp.float32)
    h = jnp.maximum(acc + b_ref[0:1, :], 0.0)
    h = h.reshape(TROW // 2, 2, FOUT).max(axis=1)
    out_ref[...] = h[None]


def _tail(z0, z1, z2, z3, wk, b2):
    zspec = pl.BlockSpec((TROW, FIN), lambda n, i: (n * NBLK + i, 0))
    return pl.pallas_call(
        _tail_body,
        grid=(NB, NBLK),
        in_specs=[
            zspec, zspec, zspec, zspec,
            pl.BlockSpec((KPOLY, FIN, FOUT), lambda n, i: (0, 0, 0)),
            pl.BlockSpec((8, FOUT), lambda n, i: (0, 0)),
        ],
        out_specs=pl.BlockSpec((1, TROW // 2, FOUT), lambda n, i: (n, i, 0)),
        out_shape=jax.ShapeDtypeStruct((NB, M // 2, FOUT), jnp.float32),
    )(z0, z1, z2, z3, wk, b2)


def kernel(x, edge_index, edge_vals, kernel, bias):
    row = edge_index[0]
    col = edge_index[1]
    z0 = x.reshape(NB * M, FIN)
    z1 = _spmm_first(z0, z0, row, col, edge_vals)
    z2 = _spmm_rec(z1, z0, row, col, edge_vals)
    z3 = _spmm_rec(z2, z1, row, col, edge_vals)
    wk = kernel.reshape(FIN, KPOLY, FOUT).transpose(1, 0, 2)
    b2 = jnp.broadcast_to(bias.reshape(1, FOUT), (8, FOUT))
    return _tail(z0, z1, z2, z3, wk, b2)


# speedup vs baseline: 6.2985x; 2.3340x over previous
import functools

import jax
import jax.numpy as jnp
from jax import lax
from jax.experimental import pallas as pl
from jax.experimental.pallas import tpu as pltpu
from jax.experimental.pallas import tpu_sc as plsc

M = 10000
FIN = 128
FOUT = 128
KPOLY = 4
E = 320000
NB = 2

NT = 16
CH = 128
NCH = E // CH
CPT = NCH // NT
NPAIR = CPT // 2
XTRA = NCH - CPT * NT
BLK = 12
EBLK = BLK * CH
RCH = 40
NRC = M // RCH
RR = -(-NRC // NT)

_LANES = 16
_FSL = FIN // _LANES


def _make_spmm(alpha, beta, gamma):
    mesh = plsc.VectorSubcoreMesh(core_axis_name="c", subcore_axis_name="s")

    @functools.partial(
        pl.kernel,
        mesh=mesh,
        out_type=jax.ShapeDtypeStruct((NB * M, FIN), jnp.float32),
        scratch_types=[
            pltpu.VMEM_SHARED((M, FIN), jnp.float32),
            pltpu.VMEM((CH, FIN), jnp.float32),
            pltpu.VMEM((CH, FIN), jnp.float32),
            pltpu.VMEM((EBLK,), jnp.int32),
            pltpu.VMEM((EBLK,), jnp.int32),
            pltpu.VMEM((EBLK,), jnp.float32),
            pltpu.VMEM((CH,), jnp.int32),
            pltpu.VMEM((CH,), jnp.int32),
            pltpu.VMEM((CH,), jnp.float32),
            pltpu.VMEM((CH,), jnp.int32),
            pltpu.VMEM((CH,), jnp.int32),
            pltpu.VMEM((CH,), jnp.float32),
            pltpu.VMEM((RCH, FIN), jnp.float32),
            pltpu.SemaphoreType.DMA,
            pltpu.SemaphoreType.DMA,
            pltpu.SemaphoreType.DMA,
            pltpu.SemaphoreType.DMA,
        ],
    )
    def spmm(z_hbm, w_hbm, row_hbm, col_hbm, val_hbm, out_hbm,
             acc_sh, rows_a, rows_b, brow, bcol, bval,
             ridx_a, gidx_a, val_a, ridx_b, gidx_b, val_b, wb,
             gsem_a, gsem_b, ssem_a, ssem_b):
        c = lax.axis_index("c")
        s = lax.axis_index("s")
        cM = c * M
        offvec = jnp.full((_LANES,), cM, jnp.int32)
        _CSL = CH // _LANES

        zvec = jnp.zeros((_LANES,), jnp.float32)

        def zero_body(r, _):
            for j in range(_FSL):
                rows_a[r, pl.ds(_LANES * j, _LANES)] = zvec
            return 0

        lax.fori_loop(0, RCH, zero_body, 0)

        def zero_chunk(i, _):
            j = s + NT * i

            @pl.when(j < NRC)
            def _():
                r0 = pl.multiple_of(j * RCH, 8)
                pltpu.sync_copy(rows_a.at[pl.ds(0, RCH)],
                                acc_sh.at[pl.ds(r0, RCH)])

            return 0

        lax.fori_loop(0, RR, zero_chunk, 0)
        plsc.subcore_barrier()

        def load_block(kg):
            e0 = pl.multiple_of(kg * CH, 8)
            pltpu.sync_copy(row_hbm.at[pl.ds(e0, EBLK)], brow)
            pltpu.sync_copy(col_hbm.at[pl.ds(e0, EBLK)], bcol)
            pltpu.sync_copy(val_hbm.at[pl.ds(e0, EBLK)], bval)

        def copy_idx(ridx, gidx, val, p):
            base = p * CH
            for g in range(_CSL):
                src = pl.ds(base + _LANES * g, _LANES)
                dst = pl.ds(_LANES * g, _LANES)
                ridx[dst] = brow[src]
                gidx[dst] = bcol[src] + offvec
                val[dst] = bval[src]

        def scale(rows, val):
            def group_body(g, _):
                vals16 = val[pl.ds(_LANES * g, _LANES)]
                for l in range(_LANES):
                    sv = lax.gather(
                        vals16,
                        jnp.full((_LANES, 1), l, jnp.int32),
                        lax.GatherDimensionNumbers(
                            offset_dims=(), collapsed_slice_dims=(0,),
                            start_index_map=(0,)),
                        slice_sizes=(1,),
                        mode=lax.GatherScatterMode.PROMISE_IN_BOUNDS)
                    e = g * _LANES + l
                    for j in range(_FSL):
                        sl = pl.ds(_LANES * j, _LANES)
                        rows[e, sl] = rows[e, sl] * sv
                return 0

            lax.fori_loop(0, _CSL, group_body, 0)

        def fire_gather(gidx, rows, sem):
            return pltpu.async_copy(z_hbm.at[gidx], rows, sem)

        def fire_scatter(rows, ridx, sem):
            return pltpu.async_copy(rows, acc_sh.at[ridx], sem, add=True)

        k0 = s * CPT
        load_block(k0)
        copy_idx(ridx_a, gidx_a, val_a, 0)
        fire_gather(gidx_a, rows_a, gsem_a)

        def pair_body(it, _):
            a = 2 * it

            @pl.when(it > 0)
            def _():
                pltpu.make_async_copy(
                    rows_b, acc_sh.at[ridx_b], ssem_b).wait()

            copy_idx(ridx_b, gidx_b, val_b, lax.rem(a, BLK) + 1)
            hgb = fire_gather(gidx_b, rows_b, gsem_b)

            pltpu.make_async_copy(z_hbm.at[gidx_a], rows_a, gsem_a).wait()
            scale(rows_a, val_a)
            hsa = fire_scatter(rows_a, ridx_a, ssem_a)

            hgb.wait()
            scale(rows_b, val_b)
            fire_scatter(rows_b, ridx_b, ssem_b)

            hsa.wait()

            @pl.when(it < NPAIR - 1)
            def _():
                nxt = a + 2

                @pl.when(lax.rem(nxt, BLK) == 0)
                def _():
                    load_block(k0 + nxt)

                copy_idx(ridx_a, gidx_a, val_a, lax.rem(nxt, BLK))
                fire_gather(gidx_a, rows_a, gsem_a)

            return 0

        lax.fori_loop(0, NPAIR, pair_body, 0)
        pltpu.make_async_copy(rows_b, acc_sh.at[ridx_b], ssem_b).wait()

        @pl.when(s < XTRA)
        def _():
            e0 = pl.multiple_of((NT * CPT + s) * CH, 8)
            pltpu.sync_copy(row_hbm.at[pl.ds(e0, CH)], brow.at[pl.ds(0, CH)])
            pltpu.sync_copy(col_hbm.at[pl.ds(e0, CH)], bcol.at[pl.ds(0, CH)])
            pltpu.sync_copy(val_hbm.at[pl.ds(e0, CH)], bval.at[pl.ds(0, CH)])
            copy_idx(ridx_a, gidx_a, val_a, 0)
            fire_gather(gidx_a, rows_a, gsem_a).wait()
            scale(rows_a, val_a)
            fire_scatter(rows_a, ridx_a, ssem_a).wait()

        plsc.subcore_barrier()

        def epi_chunk(i, _):
            jc = s + NT * i

            @pl.when(jc < NRC)
            def _():
                r0 = pl.multiple_of(jc * RCH, 8)
                g0 = pl.multiple_of(cM + r0, 8)
                pltpu.sync_copy(acc_sh.at[pl.ds(r0, RCH)],
                                rows_a.at[pl.ds(0, RCH)])
                pltpu.sync_copy(z_hbm.at[pl.ds(g0, RCH)],
                                rows_b.at[pl.ds(0, RCH)])
                if gamma != 0.0:
                    pltpu.sync_copy(w_hbm.at[pl.ds(g0, RCH)], wb)

                def comb_body(r, _):
                    for j in range(_FSL):
                        sl = pl.ds(_LANES * j, _LANES)
                        res = alpha * rows_a[r, sl] + beta * rows_b[r, sl]
                        if gamma != 0.0:
                            res = res + gamma * wb[r, sl]
                        rows_a[r, sl] = res
                    return 0

                lax.fori_loop(0, RCH, comb_body, 0)
                pltpu.sync_copy(rows_a.at[pl.ds(0, RCH)],
                                out_hbm.at[pl.ds(g0, RCH)])

            return 0

        lax.fori_loop(0, RR, epi_chunk, 0)

    return spmm


_spmm_first = _make_spmm(1.0, -1.0, 0.0)
_spmm_rec = _make_spmm(2.0, -2.0, -1.0)


TROW = 2000
NBLK = M // TROW


def _tail_body(z0_ref, z1_ref, z2_ref, z3_ref, w_ref, b_ref, out_ref):
    acc = jnp.dot(z0_ref[...], w_ref[0], preferred_element_type=jnp.float32)
    acc += jnp.dot(z1_ref[...], w_ref[1], preferred_element_type=jnp.float32)
    acc += jnp.dot(z2_ref[...], w_ref[2], preferred_element_type=jnp.float32)
    acc += jnp.dot(z3_ref[...], w_ref[3], preferred_element_type=jnp.float32)
    h = jnp.maximum(acc + b_ref[0:1, :], 0.0)
    h = h.reshape(TROW // 2, 2, FOUT).max(axis=1)
    out_ref[...] = h[None]


def _tail(z0, z1, z2, z3, wk, b2):
    zspec = pl.BlockSpec((TROW, FIN), lambda n, i: (n * NBLK + i, 0))
    return pl.pallas_call(
        _tail_body,
        grid=(NB, NBLK),
        in_specs=[
            zspec, zspec, zspec, zspec,
            pl.BlockSpec((KPOLY, FIN, FOUT), lambda n, i: (0, 0, 0)),
            pl.BlockSpec((8, FOUT), lambda n, i: (0, 0)),
        ],
        out_specs=pl.BlockSpec((1, TROW // 2, FOUT), lambda n, i: (n, i, 0)),
        out_shape=jax.ShapeDtypeStruct((NB, M // 2, FOUT), jnp.float32),
    )(z0, z1, z2, z3, wk, b2)


def kernel(x, edge_index, edge_vals, kernel, bias):
    row = edge_index[0]
    col = edge_index[1]
    z0 = x.reshape(NB * M, FIN)
    z1 = _spmm_first(z0, z0, row, col, edge_vals)
    z2 = _spmm_rec(z1, z0, row, col, edge_vals)
    z3 = _spmm_rec(z2, z1, row, col, edge_vals)
    wk = kernel.reshape(FIN, KPOLY, FOUT).transpose(1, 0, 2)
    b2 = jnp.broadcast_to(bias.reshape(1, FOUT), (8, FOUT))
    return _tail(z0, z1, z2, z3, wk, b2)


# speedup vs baseline: 6.9075x; 1.0967x over previous
import functools

import jax
import jax.numpy as jnp
from jax import lax
from jax.experimental import pallas as pl
from jax.experimental.pallas import tpu as pltpu
from jax.experimental.pallas import tpu_sc as plsc

M = 10000
FIN = 128
FOUT = 128
KPOLY = 4
E = 320000
NB = 2

NT = 16
CH = 64
NCH = E // CH
CPT = 312
NTRI = CPT // 3
XTRA = NCH - CPT * NT
BLK = 12
EBLK = BLK * CH
RCH = 40
NRC = M // RCH
RR = -(-NRC // NT)

_LANES = 16
_FSL = FIN // _LANES


def _make_spmm(alpha, beta, gamma):
    mesh = plsc.VectorSubcoreMesh(core_axis_name="c", subcore_axis_name="s")

    @functools.partial(
        pl.kernel,
        mesh=mesh,
        out_type=jax.ShapeDtypeStruct((NB * M, FIN), jnp.float32),
        scratch_types=[
            pltpu.VMEM_SHARED((M, FIN), jnp.float32),
            pltpu.VMEM((CH, FIN), jnp.float32),
            pltpu.VMEM((CH, FIN), jnp.float32),
            pltpu.VMEM((CH, FIN), jnp.float32),
            pltpu.VMEM((EBLK,), jnp.int32),
            pltpu.VMEM((EBLK,), jnp.int32),
            pltpu.VMEM((EBLK,), jnp.float32),
            pltpu.VMEM((CH,), jnp.int32),
            pltpu.VMEM((CH,), jnp.int32),
            pltpu.VMEM((CH,), jnp.float32),
            pltpu.VMEM((CH,), jnp.int32),
            pltpu.VMEM((CH,), jnp.int32),
            pltpu.VMEM((CH,), jnp.float32),
            pltpu.VMEM((CH,), jnp.int32),
            pltpu.VMEM((CH,), jnp.int32),
            pltpu.VMEM((CH,), jnp.float32),
            pltpu.VMEM((RCH, FIN), jnp.float32),
            pltpu.SemaphoreType.DMA,
            pltpu.SemaphoreType.DMA,
            pltpu.SemaphoreType.DMA,
            pltpu.SemaphoreType.DMA,
            pltpu.SemaphoreType.DMA,
            pltpu.SemaphoreType.DMA,
        ],
    )
    def spmm(z_hbm, w_hbm, row_hbm, col_hbm, val_hbm, out_hbm,
             acc_sh, rows_a, rows_b, rows_c, brow, bcol, bval,
             ridx_a, gidx_a, val_a, ridx_b, gidx_b, val_b,
             ridx_c, gidx_c, val_c, wb,
             gsem_a, gsem_b, gsem_c, ssem_a, ssem_b, ssem_c):
        c = lax.axis_index("c")
        s = lax.axis_index("s")
        cM = c * M
        offvec = jnp.full((_LANES,), cM, jnp.int32)
        _CSL = CH // _LANES

        zvec = jnp.zeros((_LANES,), jnp.float32)

        def zero_body(r, _):
            for j in range(_FSL):
                rows_a[r, pl.ds(_LANES * j, _LANES)] = zvec
            return 0

        lax.fori_loop(0, RCH, zero_body, 0)

        def zero_chunk(i, _):
            j = s + NT * i

            @pl.when(j < NRC)
            def _():
                r0 = pl.multiple_of(j * RCH, 8)
                pltpu.sync_copy(rows_a.at[pl.ds(0, RCH)],
                                acc_sh.at[pl.ds(r0, RCH)])

            return 0

        lax.fori_loop(0, RR, zero_chunk, 0)
        plsc.subcore_barrier()

        def load_block(kg):
            e0 = pl.multiple_of(kg * CH, 8)
            pltpu.sync_copy(row_hbm.at[pl.ds(e0, EBLK)], brow)
            pltpu.sync_copy(col_hbm.at[pl.ds(e0, EBLK)], bcol)
            pltpu.sync_copy(val_hbm.at[pl.ds(e0, EBLK)], bval)

        def copy_idx(ridx, gidx, val, p):
            base = p * CH
            for g in range(_CSL):
                src = pl.ds(base + _LANES * g, _LANES)
                dst = pl.ds(_LANES * g, _LANES)
                ridx[dst] = brow[src]
                gidx[dst] = bcol[src] + offvec
                val[dst] = bval[src]

        def scale(rows, val):
            def group_body(g, _):
                vals16 = val[pl.ds(_LANES * g, _LANES)]
                for l in range(_LANES):
                    sv = lax.gather(
                        vals16,
                        jnp.full((_LANES, 1), l, jnp.int32),
                        lax.GatherDimensionNumbers(
                            offset_dims=(), collapsed_slice_dims=(0,),
                            start_index_map=(0,)),
                        slice_sizes=(1,),
                        mode=lax.GatherScatterMode.PROMISE_IN_BOUNDS)
                    e = g * _LANES + l
                    for j in range(_FSL):
                        sl = pl.ds(_LANES * j, _LANES)
                        rows[e, sl] = rows[e, sl] * sv
                return 0

            lax.fori_loop(0, _CSL, group_body, 0)

        def fire_gather(gidx, rows, sem):
            return pltpu.async_copy(z_hbm.at[gidx], rows, sem)

        def fire_scatter(rows, ridx, sem):
            return pltpu.async_copy(rows, acc_sh.at[ridx], sem, add=True)

        slots = ((rows_a, ridx_a, gidx_a, val_a, gsem_a, ssem_a),
                 (rows_b, ridx_b, gidx_b, val_b, gsem_b, ssem_b),
                 (rows_c, ridx_c, gidx_c, val_c, gsem_c, ssem_c))

        k0 = s * CPT
        load_block(k0)
        copy_idx(ridx_a, gidx_a, val_a, 0)
        fire_gather(gidx_a, rows_a, gsem_a)

        def chunk_step(it, x):
            k = 3 * it + x
            rows_x, ridx_x, gidx_x, val_x, gsem_x, ssem_x = slots[x]
            rows_y, ridx_y, gidx_y, val_y, gsem_y, ssem_y = slots[(x + 1) % 3]

            @pl.when(k >= 2)
            def _():
                pltpu.make_async_copy(
                    rows_y, acc_sh.at[ridx_y], ssem_y).wait()

            @pl.when(k + 1 < CPT)
            def _():
                if x == 2:
                    @pl.when(lax.rem(k + 1, BLK) == 0)
                    def _():
                        load_block(k0 + k + 1)

                copy_idx(ridx_y, gidx_y, val_y, lax.rem(k + 1, BLK))
                fire_gather(gidx_y, rows_y, gsem_y)

            pltpu.make_async_copy(z_hbm.at[gidx_x], rows_x, gsem_x).wait()
            scale(rows_x, val_x)
            fire_scatter(rows_x, ridx_x, ssem_x)

        def tri_body(it, _):
            chunk_step(it, 0)
            chunk_step(it, 1)
            chunk_step(it, 2)
            return 0

        lax.fori_loop(0, NTRI, tri_body, 0)
        pltpu.make_async_copy(rows_b, acc_sh.at[ridx_b], ssem_b).wait()
        pltpu.make_async_copy(rows_c, acc_sh.at[ridx_c], ssem_c).wait()

        @pl.when(s < XTRA)
        def _():
            e0 = pl.multiple_of((NT * CPT + s) * CH, 8)
            pltpu.sync_copy(row_hbm.at[pl.ds(e0, CH)], brow.at[pl.ds(0, CH)])
            pltpu.sync_copy(col_hbm.at[pl.ds(e0, CH)], bcol.at[pl.ds(0, CH)])
            pltpu.sync_copy(val_hbm.at[pl.ds(e0, CH)], bval.at[pl.ds(0, CH)])
            copy_idx(ridx_a, gidx_a, val_a, 0)
            fire_gather(gidx_a, rows_a, gsem_a).wait()
            scale(rows_a, val_a)
            fire_scatter(rows_a, ridx_a, ssem_a).wait()

        plsc.subcore_barrier()

        def epi_chunk(i, _):
            jc = s + NT * i

            @pl.when(jc < NRC)
            def _():
                r0 = pl.multiple_of(jc * RCH, 8)
                g0 = pl.multiple_of(cM + r0, 8)
                pltpu.sync_copy(acc_sh.at[pl.ds(r0, RCH)],
                                rows_a.at[pl.ds(0, RCH)])
                pltpu.sync_copy(z_hbm.at[pl.ds(g0, RCH)],
                                rows_b.at[pl.ds(0, RCH)])
                if gamma != 0.0:
                    pltpu.sync_copy(w_hbm.at[pl.ds(g0, RCH)], wb)

                def comb_body(r, _):
                    for j in range(_FSL):
                        sl = pl.ds(_LANES * j, _LANES)
                        res = alpha * rows_a[r, sl] + beta * rows_b[r, sl]
                        if gamma != 0.0:
                            res = res + gamma * wb[r, sl]
                        rows_a[r, sl] = res
                    return 0

                lax.fori_loop(0, RCH, comb_body, 0)
                pltpu.sync_copy(rows_a.at[pl.ds(0, RCH)],
                                out_hbm.at[pl.ds(g0, RCH)])

            return 0

        lax.fori_loop(0, RR, epi_chunk, 0)

    return spmm


_spmm_first = _make_spmm(1.0, -1.0, 0.0)
_spmm_rec = _make_spmm(2.0, -2.0, -1.0)


TROW = 2000
NBLK = M // TROW


def _tail_body(z0_ref, z1_ref, z2_ref, z3_ref, w_ref, b_ref, out_ref):
    acc = jnp.dot(z0_ref[...], w_ref[0], preferred_element_type=jnp.float32)
    acc += jnp.dot(z1_ref[...], w_ref[1], preferred_element_type=jnp.float32)
    acc += jnp.dot(z2_ref[...], w_ref[2], preferred_element_type=jnp.float32)
    acc += jnp.dot(z3_ref[...], w_ref[3], preferred_element_type=jnp.float32)
    h = jnp.maximum(acc + b_ref[0:1, :], 0.0)
    h = h.reshape(TROW // 2, 2, FOUT).max(axis=1)
    out_ref[...] = h[None]


def _tail(z0, z1, z2, z3, wk, b2):
    zspec = pl.BlockSpec((TROW, FIN), lambda n, i: (n * NBLK + i, 0))
    return pl.pallas_call(
        _tail_body,
        grid=(NB, NBLK),
        in_specs=[
            zspec, zspec, zspec, zspec,
            pl.BlockSpec((KPOLY, FIN, FOUT), lambda n, i: (0, 0, 0)),
            pl.BlockSpec((8, FOUT), lambda n, i: (0, 0)),
        ],
        out_specs=pl.BlockSpec((1, TROW // 2, FOUT), lambda n, i: (n, i, 0)),
        out_shape=jax.ShapeDtypeStruct((NB, M // 2, FOUT), jnp.float32),
    )(z0, z1, z2, z3, wk, b2)


def kernel(x, edge_index, edge_vals, kernel, bias):
    row = edge_index[0]
    col = edge_index[1]
    z0 = x.reshape(NB * M, FIN)
    z1 = _spmm_first(z0, z0, row, col, edge_vals)
    z2 = _spmm_rec(z1, z0, row, col, edge_vals)
    z3 = _spmm_rec(z2, z1, row, col, edge_vals)
    wk = kernel.reshape(FIN, KPOLY, FOUT).transpose(1, 0, 2)
    b2 = jnp.broadcast_to(bias.reshape(1, FOUT), (8, FOUT))
    return _tail(z0, z1, z2, z3, wk, b2)


# speedup vs baseline: 6.9132x; 1.0008x over previous
import functools

import jax
import jax.numpy as jnp
from jax import lax
from jax.experimental import pallas as pl
from jax.experimental.pallas import tpu as pltpu
from jax.experimental.pallas import tpu_sc as plsc

M = 10000
FIN = 128
FOUT = 128
KPOLY = 4
E = 320000
NB = 2

NT = 16
CH = 64
NCH = E // CH
CPT = 312
NTRI = CPT // 3
XTRA = NCH - CPT * NT
BLK = 12
EBLK = BLK * CH
RCH = 40
NRC = M // RCH
RR = -(-NRC // NT)

_LANES = 16
_FSL = FIN // _LANES


def _make_spmm(alpha, beta, gamma):
    mesh = plsc.VectorSubcoreMesh(core_axis_name="c", subcore_axis_name="s")

    @functools.partial(
        pl.kernel,
        mesh=mesh,
        out_type=jax.ShapeDtypeStruct((NB * M, FIN), jnp.float32),
        scratch_types=[
            pltpu.VMEM_SHARED((M, FIN), jnp.float32),
            pltpu.VMEM((CH, FIN), jnp.float32),
            pltpu.VMEM((CH, FIN), jnp.float32),
            pltpu.VMEM((CH, FIN), jnp.float32),
            pltpu.VMEM((EBLK,), jnp.int32),
            pltpu.VMEM((EBLK,), jnp.int32),
            pltpu.VMEM((EBLK,), jnp.float32),
            pltpu.VMEM((CH,), jnp.int32),
            pltpu.VMEM((CH,), jnp.int32),
            pltpu.VMEM((CH,), jnp.float32),
            pltpu.VMEM((CH,), jnp.int32),
            pltpu.VMEM((CH,), jnp.int32),
            pltpu.VMEM((CH,), jnp.float32),
            pltpu.VMEM((CH,), jnp.int32),
            pltpu.VMEM((CH,), jnp.int32),
            pltpu.VMEM((CH,), jnp.float32),
            pltpu.VMEM((RCH, FIN), jnp.float32),
            pltpu.VMEM((RCH, FIN), jnp.float32),
            pltpu.VMEM((RCH, FIN), jnp.float32),
            pltpu.VMEM((RCH, FIN), jnp.float32),
            pltpu.SemaphoreType.DMA,
            pltpu.SemaphoreType.DMA,
            pltpu.SemaphoreType.DMA,
            pltpu.SemaphoreType.DMA,
            pltpu.SemaphoreType.DMA,
            pltpu.SemaphoreType.DMA,
            pltpu.SemaphoreType.DMA,
            pltpu.SemaphoreType.DMA,
            pltpu.SemaphoreType.DMA,
            pltpu.SemaphoreType.DMA,
            pltpu.SemaphoreType.DMA,
        ],
    )
    def spmm(z_hbm, w_hbm, row_hbm, col_hbm, val_hbm, out_hbm,
             acc_sh, rows_a, rows_b, rows_c, brow, bcol, bval,
             ridx_a, gidx_a, val_a, ridx_b, gidx_b, val_b,
             ridx_c, gidx_c, val_c, wb0, eacc1, ez1, wb1,
             gsem_a, gsem_b, gsem_c, ssem_a, ssem_b, ssem_c,
             zsem, ein0, ein1, eout0, eout1):
        c = lax.axis_index("c")
        s = lax.axis_index("s")
        cM = c * M
        offvec = jnp.full((_LANES,), cM, jnp.int32)
        _CSL = CH // _LANES

        zvec = jnp.zeros((_LANES,), jnp.float32)

        def zero_body(r, _):
            for j in range(_FSL):
                rows_a[r, pl.ds(_LANES * j, _LANES)] = zvec
            return 0

        lax.fori_loop(0, RCH, zero_body, 0)

        def zero_chunk(i, _):
            j = s + NT * i

            @pl.when(j < NRC)
            def _():
                r0 = pl.multiple_of(j * RCH, 8)
                pltpu.sync_copy(rows_a.at[pl.ds(0, RCH)],
                                acc_sh.at[pl.ds(r0, RCH)])

            return 0

        lax.fori_loop(0, RR, zero_chunk, 0)
        plsc.subcore_barrier()

        def load_block(kg):
            e0 = pl.multiple_of(kg * CH, 8)
            pltpu.sync_copy(row_hbm.at[pl.ds(e0, EBLK)], brow)
            pltpu.sync_copy(col_hbm.at[pl.ds(e0, EBLK)], bcol)
            pltpu.sync_copy(val_hbm.at[pl.ds(e0, EBLK)], bval)

        def copy_idx(ridx, gidx, val, p):
            base = p * CH
            for g in range(_CSL):
                src = pl.ds(base + _LANES * g, _LANES)
                dst = pl.ds(_LANES * g, _LANES)
                ridx[dst] = brow[src]
                gidx[dst] = bcol[src] + offvec
                val[dst] = bval[src]

        def scale(rows, val):
            def group_body(g, _):
                vals16 = val[pl.ds(_LANES * g, _LANES)]
                for l in range(_LANES):
                    sv = lax.gather(
                        vals16,
                        jnp.full((_LANES, 1), l, jnp.int32),
                        lax.GatherDimensionNumbers(
                            offset_dims=(), collapsed_slice_dims=(0,),
                            start_index_map=(0,)),
                        slice_sizes=(1,),
                        mode=lax.GatherScatterMode.PROMISE_IN_BOUNDS)
                    e = g * _LANES + l
                    for j in range(_FSL):
                        sl = pl.ds(_LANES * j, _LANES)
                        rows[e, sl] = rows[e, sl] * sv
                return 0

            lax.fori_loop(0, _CSL, group_body, 0)

        def fire_gather(gidx, rows, sem):
            return pltpu.async_copy(z_hbm.at[gidx], rows, sem)

        def fire_scatter(rows, ridx, sem):
            return pltpu.async_copy(rows, acc_sh.at[ridx], sem, add=True)

        slots = ((rows_a, ridx_a, gidx_a, val_a, gsem_a, ssem_a),
                 (rows_b, ridx_b, gidx_b, val_b, gsem_b, ssem_b),
                 (rows_c, ridx_c, gidx_c, val_c, gsem_c, ssem_c))

        k0 = s * CPT
        load_block(k0)
        copy_idx(ridx_a, gidx_a, val_a, 0)
        fire_gather(gidx_a, rows_a, gsem_a)

        def chunk_step(it, x):
            k = 3 * it + x
            rows_x, ridx_x, gidx_x, val_x, gsem_x, ssem_x = slots[x]
            rows_y, ridx_y, gidx_y, val_y, gsem_y, ssem_y = slots[(x + 1) % 3]

            @pl.when(k >= 2)
            def _():
                pltpu.make_async_copy(
                    rows_y, acc_sh.at[ridx_y], ssem_y).wait()

            @pl.when(k + 1 < CPT)
            def _():
                if x == 2:
                    @pl.when(lax.rem(k + 1, BLK) == 0)
                    def _():
                        load_block(k0 + k + 1)

                copy_idx(ridx_y, gidx_y, val_y, lax.rem(k + 1, BLK))
                fire_gather(gidx_y, rows_y, gsem_y)

            pltpu.make_async_copy(z_hbm.at[gidx_x], rows_x, gsem_x).wait()
            scale(rows_x, val_x)
            fire_scatter(rows_x, ridx_x, ssem_x)

        def tri_body(it, _):
            chunk_step(it, 0)
            chunk_step(it, 1)
            chunk_step(it, 2)
            return 0

        lax.fori_loop(0, NTRI, tri_body, 0)
        pltpu.make_async_copy(rows_b, acc_sh.at[ridx_b], ssem_b).wait()
        pltpu.make_async_copy(rows_c, acc_sh.at[ridx_c], ssem_c).wait()

        @pl.when(s < XTRA)
        def _():
            e0 = pl.multiple_of((NT * CPT + s) * CH, 8)
            pltpu.sync_copy(row_hbm.at[pl.ds(e0, CH)], brow.at[pl.ds(0, CH)])
            pltpu.sync_copy(col_hbm.at[pl.ds(e0, CH)], bcol.at[pl.ds(0, CH)])
            pltpu.sync_copy(val_hbm.at[pl.ds(e0, CH)], bval.at[pl.ds(0, CH)])
            copy_idx(ridx_a, gidx_a, val_a, 0)
            fire_gather(gidx_a, rows_a, gsem_a).wait()
            scale(rows_a, val_a)
            fire_scatter(rows_a, ridx_a, ssem_a).wait()

        plsc.subcore_barrier()

        eslots = (
            (rows_a.at[pl.ds(0, RCH)], rows_a,
             rows_b.at[pl.ds(0, RCH)], rows_b, wb0, ein0, eout0),
            (eacc1, eacc1, ez1, ez1, wb1, ein1, eout1),
        )

        def epi_offsets(ri):
            jc = s + NT * ri
            r0 = pl.multiple_of(jc * RCH, 8)
            g0 = pl.multiple_of(cM + r0, 8)
            return jc, r0, g0

        def epi_fire_in(q, ri):
            accd, _, zd, _, wbq, einq, _ = eslots[q]
            jc, r0, g0 = epi_offsets(ri)

            @pl.when(jc < NRC)
            def _():
                pltpu.async_copy(acc_sh.at[pl.ds(r0, RCH)], accd, einq)
                pltpu.async_copy(z_hbm.at[pl.ds(g0, RCH)], zd, einq)
                if gamma != 0.0:
                    pltpu.async_copy(w_hbm.at[pl.ds(g0, RCH)], wbq, einq)

        def epi_process(q, ri):
            accd, accv, zd, zv, wbq, einq, eoutq = eslots[q]
            jc, r0, g0 = epi_offsets(ri)

            @pl.when(jc < NRC)
            def _():
                pltpu.make_async_copy(
                    acc_sh.at[pl.ds(r0, RCH)], accd, einq).wait()
                pltpu.make_async_copy(
                    z_hbm.at[pl.ds(g0, RCH)], zd, einq).wait()
                if gamma != 0.0:
                    pltpu.make_async_copy(
                        w_hbm.at[pl.ds(g0, RCH)], wbq, einq).wait()

                def comb_body(r, _):
                    for j in range(_FSL):
                        sl = pl.ds(_LANES * j, _LANES)
                        res = alpha * accv[r, sl] + beta * zv[r, sl]
                        if gamma != 0.0:
                            res = res + gamma * wbq[r, sl]
                        accv[r, sl] = res
                    return 0

                lax.fori_loop(0, RCH, comb_body, 0)
                pltpu.async_copy(accd, out_hbm.at[pl.ds(g0, RCH)], eoutq)

        def epi_wait_out(q, ri):
            accd, _, _, _, _, _, eoutq = eslots[q]
            jc, r0, g0 = epi_offsets(ri)

            @pl.when(jc < NRC)
            def _():
                pltpu.make_async_copy(
                    accd, out_hbm.at[pl.ds(g0, RCH)], eoutq).wait()

        def epi_chunk(i, _):
            jc, r0, g0 = epi_offsets(i)

            @pl.when(jc < NRC)
            def _():
                pltpu.sync_copy(acc_sh.at[pl.ds(r0, RCH)],
                                rows_a.at[pl.ds(0, RCH)])
                pltpu.sync_copy(z_hbm.at[pl.ds(g0, RCH)],
                                rows_b.at[pl.ds(0, RCH)])
                if gamma != 0.0:
                    pltpu.sync_copy(w_hbm.at[pl.ds(g0, RCH)], wb0)

                def comb_body(r, _):
                    for j in range(_FSL):
                        sl = pl.ds(_LANES * j, _LANES)
                        res = alpha * rows_a[r, sl] + beta * rows_b[r, sl]
                        if gamma != 0.0:
                            res = res + gamma * wb0[r, sl]
                        rows_a[r, sl] = res
                    return 0

                lax.fori_loop(0, RCH, comb_body, 0)
                pltpu.sync_copy(rows_a.at[pl.ds(0, RCH)],
                                out_hbm.at[pl.ds(g0, RCH)])

            return 0

        lax.fori_loop(0, RR, epi_chunk, 0)

    return spmm


_spmm_first = _make_spmm(1.0, -1.0, 0.0)
_spmm_rec = _make_spmm(2.0, -2.0, -1.0)


TROW = 2000
NBLK = M // TROW


def _tail_body(z0_ref, z1_ref, z2_ref, z3_ref, w_ref, b_ref, out_ref):
    acc = jnp.dot(z0_ref[...], w_ref[0], preferred_element_type=jnp.float32)
    acc += jnp.dot(z1_ref[...], w_ref[1], preferred_element_type=jnp.float32)
    acc += jnp.dot(z2_ref[...], w_ref[2], preferred_element_type=jnp.float32)
    acc += jnp.dot(z3_ref[...], w_ref[3], preferred_element_type=jnp.float32)
    h = jnp.maximum(acc + b_ref[0:1, :], 0.0)
    h = h.reshape(TROW // 2, 2, FOUT).max(axis=1)
    out_ref[...] = h[None]


def _tail(z0, z1, z2, z3, wk, b2):
    zspec = pl.BlockSpec((TROW, FIN), lambda n, i: (n * NBLK + i, 0))
    return pl.pallas_call(
        _tail_body,
        grid=(NB, NBLK),
        in_specs=[
            zspec, zspec, zspec, zspec,
            pl.BlockSpec((KPOLY, FIN, FOUT), lambda n, i: (0, 0, 0)),
            pl.BlockSpec((8, FOUT), lambda n, i: (0, 0)),
        ],
        out_specs=pl.BlockSpec((1, TROW // 2, FOUT), lambda n, i: (n, i, 0)),
        out_shape=jax.ShapeDtypeStruct((NB, M // 2, FOUT), jnp.float32),
    )(z0, z1, z2, z3, wk, b2)


def kernel(x, edge_index, edge_vals, kernel, bias):
    row = edge_index[0]
    col = edge_index[1]
    z0 = x.reshape(NB * M, FIN)
    z1 = _spmm_first(z0, z0, row, col, edge_vals)
    z2 = _spmm_rec(z1, z0, row, col, edge_vals)
    z3 = _spmm_rec(z2, z1, row, col, edge_vals)
    wk = kernel.reshape(FIN, KPOLY, FOUT).transpose(1, 0, 2)
    b2 = jnp.broadcast_to(bias.reshape(1, FOUT), (8, FOUT))
    return _tail(z0, z1, z2, z3, wk, b2)


# speedup vs baseline: 7.3354x; 1.0611x over previous
import functools

import jax
import jax.numpy as jnp
from jax import lax
from jax.experimental import pallas as pl
from jax.experimental.pallas import tpu as pltpu
from jax.experimental.pallas import tpu_sc as plsc

M = 10000
FIN = 128
FOUT = 128
KPOLY = 4
E = 320000
NB = 2

NT = 16
CH = 64
NCH = E // CH
CPT = 312
NTRI = CPT // 3
XTRA = NCH - CPT * NT
BLK = 12
EBLK = BLK * CH
RCH = 40
NRC = M // RCH
RR = -(-NRC // NT)

_LANES = 16
_FSL = FIN // _LANES


def _make_spmm(alpha, beta, gamma):
    mesh = plsc.VectorSubcoreMesh(core_axis_name="c", subcore_axis_name="s")

    @functools.partial(
        pl.kernel,
        mesh=mesh,
        out_type=jax.ShapeDtypeStruct((NB * M, FIN), jnp.float32),
        scratch_types=[
            pltpu.VMEM_SHARED((M, FIN), jnp.float32),
            pltpu.VMEM((CH, FIN), jnp.float32),
            pltpu.VMEM((CH, FIN), jnp.float32),
            pltpu.VMEM((CH, FIN), jnp.float32),
            pltpu.VMEM((EBLK,), jnp.int32),
            pltpu.VMEM((EBLK,), jnp.int32),
            pltpu.VMEM((EBLK,), jnp.float32),
            pltpu.VMEM((CH,), jnp.int32),
            pltpu.VMEM((CH,), jnp.int32),
            pltpu.VMEM((CH,), jnp.float32),
            pltpu.VMEM((CH,), jnp.int32),
            pltpu.VMEM((CH,), jnp.int32),
            pltpu.VMEM((CH,), jnp.float32),
            pltpu.VMEM((CH,), jnp.int32),
            pltpu.VMEM((CH,), jnp.int32),
            pltpu.VMEM((CH,), jnp.float32),
            pltpu.VMEM((RCH, FIN), jnp.float32),
            pltpu.VMEM((RCH, FIN), jnp.float32),
            pltpu.VMEM((RCH, FIN), jnp.float32),
            pltpu.VMEM((RCH, FIN), jnp.float32),
            pltpu.VMEM((RCH,), jnp.int32),
            pltpu.VMEM((RCH,), jnp.int32),
            pltpu.VMEM((RCH,), jnp.int32),
            pltpu.VMEM((RCH,), jnp.int32),
            pltpu.SemaphoreType.DMA,
            pltpu.SemaphoreType.DMA,
            pltpu.SemaphoreType.DMA,
            pltpu.SemaphoreType.DMA,
            pltpu.SemaphoreType.DMA,
            pltpu.SemaphoreType.DMA,
            pltpu.SemaphoreType.DMA,
            pltpu.SemaphoreType.DMA,
            pltpu.SemaphoreType.DMA,
            pltpu.SemaphoreType.DMA,
            pltpu.SemaphoreType.DMA,
        ],
    )
    def spmm(z_hbm, w_hbm, row_hbm, col_hbm, val_hbm, out_hbm,
             acc_sh, rows_a, rows_b, rows_c, brow, bcol, bval,
             ridx_a, gidx_a, val_a, ridx_b, gidx_b, val_b,
             ridx_c, gidx_c, val_c, wb0, eacc1, ez1, wb1,
             ia0, ih0, ia1, ih1,
             gsem_a, gsem_b, gsem_c, ssem_a, ssem_b, ssem_c,
             zsem, ein0, ein1, eout0, eout1):
        c = lax.axis_index("c")
        s = lax.axis_index("s")
        cM = c * M
        offvec = jnp.full((_LANES,), cM, jnp.int32)
        _CSL = CH // _LANES

        zvec = jnp.zeros((_LANES,), jnp.float32)

        def zero_body(r, _):
            for j in range(_FSL):
                rows_a[r, pl.ds(_LANES * j, _LANES)] = zvec
            return 0

        lax.fori_loop(0, RCH, zero_body, 0)

        def zero_chunk(i, _):
            j = s + NT * i

            @pl.when(j < NRC)
            def _():
                r0 = pl.multiple_of(j * RCH, 8)
                pltpu.sync_copy(rows_a.at[pl.ds(0, RCH)],
                                acc_sh.at[pl.ds(r0, RCH)])

            return 0

        lax.fori_loop(0, RR, zero_chunk, 0)
        plsc.subcore_barrier()

        def load_block(kg):
            e0 = pl.multiple_of(kg * CH, 8)
            pltpu.sync_copy(row_hbm.at[pl.ds(e0, EBLK)], brow)
            pltpu.sync_copy(col_hbm.at[pl.ds(e0, EBLK)], bcol)
            pltpu.sync_copy(val_hbm.at[pl.ds(e0, EBLK)], bval)

        def copy_idx(ridx, gidx, val, p):
            base = p * CH
            for g in range(_CSL):
                src = pl.ds(base + _LANES * g, _LANES)
                dst = pl.ds(_LANES * g, _LANES)
                ridx[dst] = brow[src]
                gidx[dst] = bcol[src] + offvec
                val[dst] = bval[src]

        def scale(rows, val):
            def group_body(g, _):
                vals16 = val[pl.ds(_LANES * g, _LANES)]
                for l in range(_LANES):
                    sv = lax.gather(
                        vals16,
                        jnp.full((_LANES, 1), l, jnp.int32),
                        lax.GatherDimensionNumbers(
                            offset_dims=(), collapsed_slice_dims=(0,),
                            start_index_map=(0,)),
                        slice_sizes=(1,),
                        mode=lax.GatherScatterMode.PROMISE_IN_BOUNDS)
                    e = g * _LANES + l
                    for j in range(_FSL):
                        sl = pl.ds(_LANES * j, _LANES)
                        rows[e, sl] = rows[e, sl] * sv
                return 0

            lax.fori_loop(0, _CSL, group_body, 0)

        def fire_gather(gidx, rows, sem):
            return pltpu.async_copy(z_hbm.at[gidx], rows, sem)

        def fire_scatter(rows, ridx, sem):
            return pltpu.async_copy(rows, acc_sh.at[ridx], sem, add=True)

        slots = ((rows_a, ridx_a, gidx_a, val_a, gsem_a, ssem_a),
                 (rows_b, ridx_b, gidx_b, val_b, gsem_b, ssem_b),
                 (rows_c, ridx_c, gidx_c, val_c, gsem_c, ssem_c))

        k0 = s * CPT
        load_block(k0)
        copy_idx(ridx_a, gidx_a, val_a, 0)
        fire_gather(gidx_a, rows_a, gsem_a)

        def chunk_step(it, x):
            k = 3 * it + x
            rows_x, ridx_x, gidx_x, val_x, gsem_x, ssem_x = slots[x]
            rows_y, ridx_y, gidx_y, val_y, gsem_y, ssem_y = slots[(x + 1) % 3]

            @pl.when(k >= 2)
            def _():
                pltpu.make_async_copy(
                    rows_y, acc_sh.at[ridx_y], ssem_y).wait()

            @pl.when(k + 1 < CPT)
            def _():
                if x == 2:
                    @pl.when(lax.rem(k + 1, BLK) == 0)
                    def _():
                        load_block(k0 + k + 1)

                copy_idx(ridx_y, gidx_y, val_y, lax.rem(k + 1, BLK))
                fire_gather(gidx_y, rows_y, gsem_y)

            pltpu.make_async_copy(z_hbm.at[gidx_x], rows_x, gsem_x).wait()
            scale(rows_x, val_x)
            fire_scatter(rows_x, ridx_x, ssem_x)

        def tri_body(it, _):
            chunk_step(it, 0)
            chunk_step(it, 1)
            chunk_step(it, 2)
            return 0

        lax.fori_loop(0, NTRI, tri_body, 0)
        pltpu.make_async_copy(rows_b, acc_sh.at[ridx_b], ssem_b).wait()
        pltpu.make_async_copy(rows_c, acc_sh.at[ridx_c], ssem_c).wait()

        @pl.when(s < XTRA)
        def _():
            e0 = pl.multiple_of((NT * CPT + s) * CH, 8)
            pltpu.sync_copy(row_hbm.at[pl.ds(e0, CH)], brow.at[pl.ds(0, CH)])
            pltpu.sync_copy(col_hbm.at[pl.ds(e0, CH)], bcol.at[pl.ds(0, CH)])
            pltpu.sync_copy(val_hbm.at[pl.ds(e0, CH)], bval.at[pl.ds(0, CH)])
            copy_idx(ridx_a, gidx_a, val_a, 0)
            fire_gather(gidx_a, rows_a, gsem_a).wait()
            scale(rows_a, val_a)
            fire_scatter(rows_a, ridx_a, ssem_a).wait()

        plsc.subcore_barrier()

        eslots = (
            (rows_a.at[pl.ds(0, RCH)], rows_a,
             rows_b.at[pl.ds(0, RCH)], rows_b, wb0, ia0, ih0, ein0, eout0),
            (eacc1, eacc1, ez1, ez1, wb1, ia1, ih1, ein1, eout1),
        )

        def epi_offsets(ri):
            jc = s + NT * ri
            r0 = pl.multiple_of(jc * RCH, 8)
            g0 = pl.multiple_of(cM + r0, 8)
            return jc, r0, g0

        iot16 = lax.iota(jnp.int32, _LANES)

        def set_idx(buf, base):
            buf[pl.ds(0, _LANES)] = base + iot16
            buf[pl.ds(16, _LANES)] = (base + 16) + iot16
            buf[pl.ds(24, _LANES)] = (base + 24) + iot16

        def epi_fire_in(q, ri):
            _, _, zd, _, wbq, _, ihq, einq, _ = eslots[q]
            jc, r0, g0 = epi_offsets(ri)

            @pl.when(jc < NRC)
            def _():
                set_idx(ihq, g0)
                pltpu.async_copy(z_hbm.at[ihq], zd, einq)
                if gamma != 0.0:
                    pltpu.async_copy(w_hbm.at[ihq], wbq, einq)

        def epi_process(q, ri):
            _, _, zd, zv, wbq, _, ihq, einq, _ = eslots[q]
            jc, r0, g0 = epi_offsets(ri)

            @pl.when(jc < NRC)
            def _():
                pltpu.sync_copy(acc_sh.at[pl.ds(r0, RCH)],
                                rows_a.at[pl.ds(0, RCH)])
                pltpu.make_async_copy(z_hbm.at[ihq], zd, einq).wait()
                if gamma != 0.0:
                    pltpu.make_async_copy(w_hbm.at[ihq], wbq, einq).wait()

                def comb_body(r, _):
                    for j in range(_FSL):
                        sl = pl.ds(_LANES * j, _LANES)
                        res = alpha * rows_a[r, sl] + beta * zv[r, sl]
                        if gamma != 0.0:
                            res = res + gamma * wbq[r, sl]
                        rows_a[r, sl] = res
                    return 0

                lax.fori_loop(0, RCH, comb_body, 0)
                pltpu.sync_copy(rows_a.at[pl.ds(0, RCH)],
                                out_hbm.at[pl.ds(g0, RCH)])

        epi_fire_in(0, jnp.int32(0))
        epi_fire_in(1, jnp.int32(1))

        def epi_pair(t, _):
            ra = 2 * t
            rb = 2 * t + 1
            epi_process(0, ra)
            epi_fire_in(0, ra + 2)
            epi_process(1, rb)
            epi_fire_in(1, rb + 2)
            return 0

        lax.fori_loop(0, RR // 2, epi_pair, 0)

    return spmm


_spmm_first = _make_spmm(1.0, -1.0, 0.0)
_spmm_rec = _make_spmm(2.0, -2.0, -1.0)


TROW = 2000
NBLK = M // TROW


def _tail_body(z0_ref, z1_ref, z2_ref, z3_ref, w_ref, b_ref, out_ref):
    acc = jnp.dot(z0_ref[...], w_ref[0], preferred_element_type=jnp.float32)
    acc += jnp.dot(z1_ref[...], w_ref[1], preferred_element_type=jnp.float32)
    acc += jnp.dot(z2_ref[...], w_ref[2], preferred_element_type=jnp.float32)
    acc += jnp.dot(z3_ref[...], w_ref[3], preferred_element_type=jnp.float32)
    h = jnp.maximum(acc + b_ref[0:1, :], 0.0)
    h = h.reshape(TROW // 2, 2, FOUT).max(axis=1)
    out_ref[...] = h[None]


def _tail(z0, z1, z2, z3, wk, b2):
    zspec = pl.BlockSpec((TROW, FIN), lambda n, i: (n * NBLK + i, 0))
    return pl.pallas_call(
        _tail_body,
        grid=(NB, NBLK),
        in_specs=[
            zspec, zspec, zspec, zspec,
            pl.BlockSpec((KPOLY, FIN, FOUT), lambda n, i: (0, 0, 0)),
            pl.BlockSpec((8, FOUT), lambda n, i: (0, 0)),
        ],
        out_specs=pl.BlockSpec((1, TROW // 2, FOUT), lambda n, i: (n, i, 0)),
        out_shape=jax.ShapeDtypeStruct((NB, M // 2, FOUT), jnp.float32),
    )(z0, z1, z2, z3, wk, b2)


def kernel(x, edge_index, edge_vals, kernel, bias):
    row = edge_index[0]
    col = edge_index[1]
    z0 = x.reshape(NB * M, FIN)
    z1 = _spmm_first(z0, z0, row, col, edge_vals)
    z2 = _spmm_rec(z1, z0, row, col, edge_vals)
    z3 = _spmm_rec(z2, z1, row, col, edge_vals)
    wk = kernel.reshape(FIN, KPOLY, FOUT).transpose(1, 0, 2)
    b2 = jnp.broadcast_to(bias.reshape(1, FOUT), (8, FOUT))
    return _tail(z0, z1, z2, z3, wk, b2)


# speedup vs baseline: 7.4249x; 1.0122x over previous
import functools

import jax
import jax.numpy as jnp
from jax import lax
from jax.experimental import pallas as pl
from jax.experimental.pallas import tpu as pltpu
from jax.experimental.pallas import tpu_sc as plsc

M = 10000
FIN = 128
FOUT = 128
KPOLY = 4
E = 320000
NB = 2

NT = 16
CH = 64
NCH = E // CH
CPT = 312
NTRI = CPT // 3
XTRA = NCH - CPT * NT
BLK = 12
EBLK = BLK * CH
RCH = 40
NRC = M // RCH
RR = -(-NRC // NT)

_LANES = 16
_FSL = FIN // _LANES


def _make_spmm(alpha, beta, gamma):
    mesh = plsc.VectorSubcoreMesh(core_axis_name="c", subcore_axis_name="s")

    @functools.partial(
        pl.kernel,
        mesh=mesh,
        out_type=jax.ShapeDtypeStruct((NB * M, FIN), jnp.float32),
        scratch_types=[
            pltpu.VMEM_SHARED((M, FIN), jnp.float32),
            pltpu.VMEM((CH, FIN), jnp.float32),
            pltpu.VMEM((CH, FIN), jnp.float32),
            pltpu.VMEM((CH, FIN), jnp.float32),
            pltpu.VMEM((EBLK,), jnp.int32),
            pltpu.VMEM((EBLK,), jnp.int32),
            pltpu.VMEM((EBLK,), jnp.float32),
            pltpu.VMEM((CH,), jnp.int32),
            pltpu.VMEM((CH,), jnp.int32),
            pltpu.VMEM((CH,), jnp.float32),
            pltpu.VMEM((CH,), jnp.int32),
            pltpu.VMEM((CH,), jnp.int32),
            pltpu.VMEM((CH,), jnp.float32),
            pltpu.VMEM((CH,), jnp.int32),
            pltpu.VMEM((CH,), jnp.int32),
            pltpu.VMEM((CH,), jnp.float32),
            pltpu.VMEM((RCH, FIN), jnp.float32),
            pltpu.VMEM((RCH, FIN), jnp.float32),
            pltpu.VMEM((RCH, FIN), jnp.float32),
            pltpu.VMEM((RCH, FIN), jnp.float32),
            pltpu.VMEM((RCH,), jnp.int32),
            pltpu.VMEM((RCH,), jnp.int32),
            pltpu.VMEM((RCH,), jnp.int32),
            pltpu.VMEM((RCH,), jnp.int32),
            pltpu.SemaphoreType.DMA,
            pltpu.SemaphoreType.DMA,
            pltpu.SemaphoreType.DMA,
            pltpu.SemaphoreType.DMA,
            pltpu.SemaphoreType.DMA,
            pltpu.SemaphoreType.DMA,
            pltpu.SemaphoreType.DMA,
            pltpu.SemaphoreType.DMA,
            pltpu.SemaphoreType.DMA,
            pltpu.SemaphoreType.DMA,
            pltpu.SemaphoreType.DMA,
        ],
    )
    def spmm(z_hbm, w_hbm, row_hbm, col_hbm, val_hbm, out_hbm,
             acc_sh, rows_a, rows_b, rows_c, brow, bcol, bval,
             ridx_a, gidx_a, val_a, ridx_b, gidx_b, val_b,
             ridx_c, gidx_c, val_c, wb0, eacc1, ez1, wb1,
             ia0, ih0, ia1, ih1,
             gsem_a, gsem_b, gsem_c, ssem_a, ssem_b, ssem_c,
             zsem, ein0, ein1, eout0, eout1):
        c = lax.axis_index("c")
        s = lax.axis_index("s")
        cM = c * M
        offvec = jnp.full((_LANES,), cM, jnp.int32)
        _CSL = CH // _LANES

        zvec = jnp.zeros((_LANES,), jnp.float32)

        def zero_body(r, _):
            for j in range(_FSL):
                rows_a[r, pl.ds(_LANES * j, _LANES)] = zvec
            return 0

        lax.fori_loop(0, RCH, zero_body, 0)

        def zero_chunk(i, _):
            j = s + NT * i

            @pl.when(j < NRC)
            def _():
                r0 = pl.multiple_of(j * RCH, 8)
                pltpu.sync_copy(rows_a.at[pl.ds(0, RCH)],
                                acc_sh.at[pl.ds(r0, RCH)])

            return 0

        lax.fori_loop(0, RR, zero_chunk, 0)
        plsc.subcore_barrier()

        def load_block(kg):
            e0 = pl.multiple_of(kg * CH, 8)
            pltpu.sync_copy(row_hbm.at[pl.ds(e0, EBLK)], brow)
            pltpu.sync_copy(col_hbm.at[pl.ds(e0, EBLK)], bcol)
            pltpu.sync_copy(val_hbm.at[pl.ds(e0, EBLK)], bval)

        def copy_idx(ridx, gidx, val, p):
            base = p * CH
            for g in range(_CSL):
                src = pl.ds(base + _LANES * g, _LANES)
                dst = pl.ds(_LANES * g, _LANES)
                ridx[dst] = brow[src]
                gidx[dst] = bcol[src] + offvec
                val[dst] = bval[src]

        def scale(rows, val):
            def group_body(g, _):
                vals16 = val[pl.ds(_LANES * g, _LANES)]
                for l in range(_LANES):
                    sv = lax.gather(
                        vals16,
                        jnp.full((_LANES, 1), l, jnp.int32),
                        lax.GatherDimensionNumbers(
                            offset_dims=(), collapsed_slice_dims=(0,),
                            start_index_map=(0,)),
                        slice_sizes=(1,),
                        mode=lax.GatherScatterMode.PROMISE_IN_BOUNDS)
                    e = g * _LANES + l
                    for j in range(_FSL):
                        sl = pl.ds(_LANES * j, _LANES)
                        rows[e, sl] = rows[e, sl] * sv
                return 0

            lax.fori_loop(0, _CSL, group_body, 0)

        def fire_gather(gidx, rows, sem):
            return pltpu.async_copy(z_hbm.at[gidx], rows, sem)

        def fire_scatter(rows, ridx, sem):
            return pltpu.async_copy(rows, acc_sh.at[ridx], sem, add=True)

        slots = ((rows_a, ridx_a, gidx_a, val_a, gsem_a, ssem_a),
                 (rows_b, ridx_b, gidx_b, val_b, gsem_b, ssem_b),
                 (rows_c, ridx_c, gidx_c, val_c, gsem_c, ssem_c))

        k0 = s * CPT
        load_block(k0)
        copy_idx(ridx_a, gidx_a, val_a, 0)
        fire_gather(gidx_a, rows_a, gsem_a)

        def chunk_step(it, x):
            k = 3 * it + x
            rows_x, ridx_x, gidx_x, val_x, gsem_x, ssem_x = slots[x]
            rows_y, ridx_y, gidx_y, val_y, gsem_y, ssem_y = slots[(x + 1) % 3]

            @pl.when(k >= 2)
            def _():
                pltpu.make_async_copy(
                    rows_y, acc_sh.at[ridx_y], ssem_y).wait()

            @pl.when(k + 1 < CPT)
            def _():
                if x == 2:
                    @pl.when(lax.rem(k + 1, BLK) == 0)
                    def _():
                        load_block(k0 + k + 1)

                copy_idx(ridx_y, gidx_y, val_y, lax.rem(k + 1, BLK))
                fire_gather(gidx_y, rows_y, gsem_y)

            pltpu.make_async_copy(z_hbm.at[gidx_x], rows_x, gsem_x).wait()
            scale(rows_x, val_x)
            fire_scatter(rows_x, ridx_x, ssem_x)

        def tri_body(it, _):
            chunk_step(it, 0)
            chunk_step(it, 1)
            chunk_step(it, 2)
            return 0

        lax.fori_loop(0, NTRI, tri_body, 0)
        pltpu.make_async_copy(rows_b, acc_sh.at[ridx_b], ssem_b).wait()
        pltpu.make_async_copy(rows_c, acc_sh.at[ridx_c], ssem_c).wait()

        @pl.when(s < XTRA)
        def _():
            e0 = pl.multiple_of((NT * CPT + s) * CH, 8)
            pltpu.sync_copy(row_hbm.at[pl.ds(e0, CH)], brow.at[pl.ds(0, CH)])
            pltpu.sync_copy(col_hbm.at[pl.ds(e0, CH)], bcol.at[pl.ds(0, CH)])
            pltpu.sync_copy(val_hbm.at[pl.ds(e0, CH)], bval.at[pl.ds(0, CH)])
            copy_idx(ridx_a, gidx_a, val_a, 0)
            fire_gather(gidx_a, rows_a, gsem_a).wait()
            scale(rows_a, val_a)
            fire_scatter(rows_a, ridx_a, ssem_a).wait()

        plsc.subcore_barrier()

        eslots = (
            (rows_a.at[pl.ds(0, RCH)], rows_a,
             rows_b.at[pl.ds(0, RCH)], rows_b, wb0, ia0, ih0, ein0, eout0),
            (eacc1, eacc1, ez1, ez1, wb1, ia1, ih1, ein1, eout1),
        )

        def epi_offsets(ri):
            jc = s + NT * ri
            r0 = pl.multiple_of(jc * RCH, 8)
            g0 = pl.multiple_of(cM + r0, 8)
            return jc, r0, g0

        iot16 = lax.iota(jnp.int32, _LANES)

        def set_idx(buf, base):
            buf[pl.ds(0, _LANES)] = base + iot16
            buf[pl.ds(16, _LANES)] = (base + 16) + iot16
            buf[pl.ds(24, _LANES)] = (base + 24) + iot16

        def epi_fire_in(q, ri):
            _, _, zd, _, wbq, _, ihq, einq, _ = eslots[q]
            jc, r0, g0 = epi_offsets(ri)

            @pl.when(jc < NRC)
            def _():
                set_idx(ihq, g0)
                pltpu.async_copy(z_hbm.at[ihq], zd, einq)
                if gamma != 0.0:
                    pltpu.async_copy(w_hbm.at[ihq], wbq, einq)

        eaccs = (rows_a, rows_c)

        def epi_wait_out(q, ri):
            accq = eaccs[q]
            ioq = eslots[q][5]
            jc, r0, g0 = epi_offsets(ri)

            @pl.when(jnp.logical_and(ri >= 0, jc < NRC))
            def _():
                pltpu.make_async_copy(
                    accq.at[pl.ds(0, RCH)], out_hbm.at[ioq],
                    eslots[q][8]).wait()

        def epi_process(q, ri):
            _, _, zd, zv, wbq, ioq, ihq, einq, eoutq = eslots[q]
            accq = eaccs[q]
            jc, r0, g0 = epi_offsets(ri)
            epi_wait_out(q, ri - 2)

            @pl.when(jc < NRC)
            def _():
                pltpu.sync_copy(acc_sh.at[pl.ds(r0, RCH)],
                                accq.at[pl.ds(0, RCH)])
                pltpu.make_async_copy(z_hbm.at[ihq], zd, einq).wait()
                if gamma != 0.0:
                    pltpu.make_async_copy(w_hbm.at[ihq], wbq, einq).wait()

                def comb_body(r, _):
                    for j in range(_FSL):
                        sl = pl.ds(_LANES * j, _LANES)
                        res = alpha * accq[r, sl] + beta * zv[r, sl]
                        if gamma != 0.0:
                            res = res + gamma * wbq[r, sl]
                        accq[r, sl] = res
                    return 0

                lax.fori_loop(0, RCH, comb_body, 0)
                set_idx(ioq, g0)
                pltpu.async_copy(accq.at[pl.ds(0, RCH)],
                                 out_hbm.at[ioq], eoutq)

        epi_fire_in(0, jnp.int32(0))
        epi_fire_in(1, jnp.int32(1))

        def epi_pair(t, _):
            ra = 2 * t
            rb = 2 * t + 1
            epi_process(0, ra)
            epi_fire_in(0, ra + 2)
            epi_process(1, rb)
            epi_fire_in(1, rb + 2)
            return 0

        lax.fori_loop(0, RR // 2, epi_pair, 0)
        epi_wait_out(0, jnp.int32(RR - 2))
        epi_wait_out(1, jnp.int32(RR - 1))

    return spmm


_spmm_first = _make_spmm(1.0, -1.0, 0.0)
_spmm_rec = _make_spmm(2.0, -2.0, -1.0)


TROW = 2000
NBLK = M // TROW


def _tail_body(z0_ref, z1_ref, z2_ref, z3_ref, w_ref, b_ref, out_ref):
    acc = jnp.dot(z0_ref[...], w_ref[0], preferred_element_type=jnp.float32)
    acc += jnp.dot(z1_ref[...], w_ref[1], preferred_element_type=jnp.float32)
    acc += jnp.dot(z2_ref[...], w_ref[2], preferred_element_type=jnp.float32)
    acc += jnp.dot(z3_ref[...], w_ref[3], preferred_element_type=jnp.float32)
    h = jnp.maximum(acc + b_ref[0:1, :], 0.0)
    h = h.reshape(TROW // 2, 2, FOUT).max(axis=1)
    out_ref[...] = h[None]


def _tail(z0, z1, z2, z3, wk, b2):
    zspec = pl.BlockSpec((TROW, FIN), lambda n, i: (n * NBLK + i, 0))
    return pl.pallas_call(
        _tail_body,
        grid=(NB, NBLK),
        in_specs=[
            zspec, zspec, zspec, zspec,
            pl.BlockSpec((KPOLY, FIN, FOUT), lambda n, i: (0, 0, 0)),
            pl.BlockSpec((8, FOUT), lambda n, i: (0, 0)),
        ],
        out_specs=pl.BlockSpec((1, TROW // 2, FOUT), lambda n, i: (n, i, 0)),
        out_shape=jax.ShapeDtypeStruct((NB, M // 2, FOUT), jnp.float32),
    )(z0, z1, z2, z3, wk, b2)


def kernel(x, edge_index, edge_vals, kernel, bias):
    row = edge_index[0]
    col = edge_index[1]
    z0 = x.reshape(NB * M, FIN)
    z1 = _spmm_first(z0, z0, row, col, edge_vals)
    z2 = _spmm_rec(z1, z0, row, col, edge_vals)
    z3 = _spmm_rec(z2, z1, row, col, edge_vals)
    wk = kernel.reshape(FIN, KPOLY, FOUT).transpose(1, 0, 2)
    b2 = jnp.broadcast_to(bias.reshape(1, FOUT), (8, FOUT))
    return _tail(z0, z1, z2, z3, wk, b2)


# speedup vs baseline: 7.4475x; 1.0030x over previous
import functools

import jax
import jax.numpy as jnp
from jax import lax
from jax.experimental import pallas as pl
from jax.experimental.pallas import tpu as pltpu
from jax.experimental.pallas import tpu_sc as plsc

M = 10000
FIN = 128
FOUT = 128
KPOLY = 4
E = 320000
NB = 2

NT = 16
CH = 64
NCH = E // CH
CPT = 312
NTRI = CPT // 3
XTRA = NCH - CPT * NT
BLK = 12
EBLK = BLK * CH
RCH = 40
NRC = M // RCH
RR = -(-NRC // NT)

_LANES = 16
_FSL = FIN // _LANES


def _make_spmm(alpha, beta, gamma):
    mesh = plsc.VectorSubcoreMesh(core_axis_name="c", subcore_axis_name="s")

    @functools.partial(
        pl.kernel,
        mesh=mesh,
        out_type=jax.ShapeDtypeStruct((NB * M, FIN), jnp.float32),
        scratch_types=[
            pltpu.VMEM_SHARED((M, FIN), jnp.float32),
            pltpu.VMEM((CH, FIN), jnp.float32),
            pltpu.VMEM((CH, FIN), jnp.float32),
            pltpu.VMEM((CH, FIN), jnp.float32),
            pltpu.VMEM((EBLK,), jnp.int32),
            pltpu.VMEM((EBLK,), jnp.int32),
            pltpu.VMEM((EBLK,), jnp.float32),
            pltpu.VMEM((CH,), jnp.int32),
            pltpu.VMEM((CH,), jnp.int32),
            pltpu.VMEM((CH,), jnp.float32),
            pltpu.VMEM((CH,), jnp.int32),
            pltpu.VMEM((CH,), jnp.int32),
            pltpu.VMEM((CH,), jnp.float32),
            pltpu.VMEM((CH,), jnp.int32),
            pltpu.VMEM((CH,), jnp.int32),
            pltpu.VMEM((CH,), jnp.float32),
            pltpu.VMEM((RCH, FIN), jnp.float32),
            pltpu.VMEM((RCH, FIN), jnp.float32),
            pltpu.VMEM((RCH, FIN), jnp.float32),
            pltpu.VMEM((RCH, FIN), jnp.float32),
            pltpu.VMEM((RCH,), jnp.int32),
            pltpu.VMEM((RCH,), jnp.int32),
            pltpu.VMEM((RCH,), jnp.int32),
            pltpu.VMEM((RCH,), jnp.int32),
            pltpu.SemaphoreType.DMA,
            pltpu.SemaphoreType.DMA,
            pltpu.SemaphoreType.DMA,
            pltpu.SemaphoreType.DMA,
            pltpu.SemaphoreType.DMA,
            pltpu.SemaphoreType.DMA,
            pltpu.SemaphoreType.DMA,
            pltpu.SemaphoreType.DMA,
            pltpu.SemaphoreType.DMA,
            pltpu.SemaphoreType.DMA,
            pltpu.SemaphoreType.DMA,
        ],
    )
    def spmm(z_hbm, w_hbm, row_hbm, col_hbm, val_hbm, out_hbm,
             acc_sh, rows_a, rows_b, rows_c, brow, bcol, bval,
             ridx_a, gidx_a, val_a, ridx_b, gidx_b, val_b,
             ridx_c, gidx_c, val_c, wb0, eacc1, ez1, wb1,
             ia0, ih0, ia1, ih1,
             gsem_a, gsem_b, gsem_c, ssem_a, ssem_b, ssem_c,
             zsem, ein0, ein1, eout0, eout1):
        c = lax.axis_index("c")
        s = lax.axis_index("s")
        cM = c * M
        offvec = jnp.full((_LANES,), cM, jnp.int32)
        _CSL = CH // _LANES

        iot16 = lax.iota(jnp.int32, _LANES)

        def set_idx(buf, base):
            buf[pl.ds(0, _LANES)] = base + iot16
            buf[pl.ds(16, _LANES)] = (base + 16) + iot16
            buf[pl.ds(24, _LANES)] = (base + 24) + iot16

        zvec = jnp.zeros((_LANES,), jnp.float32)

        def zero_body(r, _):
            for j in range(_FSL):
                rows_a[r, pl.ds(_LANES * j, _LANES)] = zvec
            return 0

        lax.fori_loop(0, RCH, zero_body, 0)
        zidx = (ia0, ia1)
        zsems = (ein0, ein1)

        def zfire(q, i):
            j = s + NT * i

            @pl.when(j < NRC)
            def _():
                set_idx(zidx[q], j * RCH)
                pltpu.async_copy(rows_a.at[pl.ds(0, RCH)],
                                 acc_sh.at[zidx[q]], zsems[q])

        def zwait(q, i):
            j = s + NT * i

            @pl.when(jnp.logical_and(i >= 0, j < NRC))
            def _():
                pltpu.make_async_copy(rows_a.at[pl.ds(0, RCH)],
                                      acc_sh.at[zidx[q]], zsems[q]).wait()

        def zero_pair(t, _):
            zwait(0, 2 * t - 2)
            zfire(0, 2 * t)
            zwait(1, 2 * t - 1)
            zfire(1, 2 * t + 1)
            return 0

        lax.fori_loop(0, RR // 2, zero_pair, 0)
        zwait(0, jnp.int32(RR - 2))
        zwait(1, jnp.int32(RR - 1))
        plsc.subcore_barrier()

        def load_block(kg):
            e0 = pl.multiple_of(kg * CH, 8)
            pltpu.sync_copy(row_hbm.at[pl.ds(e0, EBLK)], brow)
            pltpu.sync_copy(col_hbm.at[pl.ds(e0, EBLK)], bcol)
            pltpu.sync_copy(val_hbm.at[pl.ds(e0, EBLK)], bval)

        def copy_idx(ridx, gidx, val, p):
            base = p * CH
            for g in range(_CSL):
                src = pl.ds(base + _LANES * g, _LANES)
                dst = pl.ds(_LANES * g, _LANES)
                ridx[dst] = brow[src]
                gidx[dst] = bcol[src] + offvec
                val[dst] = bval[src]

        def scale(rows, val):
            def group_body(g, _):
                vals16 = val[pl.ds(_LANES * g, _LANES)]
                for l in range(_LANES):
                    sv = lax.gather(
                        vals16,
                        jnp.full((_LANES, 1), l, jnp.int32),
                        lax.GatherDimensionNumbers(
                            offset_dims=(), collapsed_slice_dims=(0,),
                            start_index_map=(0,)),
                        slice_sizes=(1,),
                        mode=lax.GatherScatterMode.PROMISE_IN_BOUNDS)
                    e = g * _LANES + l
                    for j in range(_FSL):
                        sl = pl.ds(_LANES * j, _LANES)
                        rows[e, sl] = rows[e, sl] * sv
                return 0

            lax.fori_loop(0, _CSL, group_body, 0)

        def fire_gather(gidx, rows, sem):
            return pltpu.async_copy(z_hbm.at[gidx], rows, sem)

        def fire_scatter(rows, ridx, sem):
            return pltpu.async_copy(rows, acc_sh.at[ridx], sem, add=True)

        slots = ((rows_a, ridx_a, gidx_a, val_a, gsem_a, ssem_a),
                 (rows_b, ridx_b, gidx_b, val_b, gsem_b, ssem_b),
                 (rows_c, ridx_c, gidx_c, val_c, gsem_c, ssem_c))

        k0 = s * CPT
        load_block(k0)
        copy_idx(ridx_a, gidx_a, val_a, 0)
        fire_gather(gidx_a, rows_a, gsem_a)

        def chunk_step(it, x):
            k = 3 * it + x
            rows_x, ridx_x, gidx_x, val_x, gsem_x, ssem_x = slots[x]
            rows_y, ridx_y, gidx_y, val_y, gsem_y, ssem_y = slots[(x + 1) % 3]

            @pl.when(k >= 2)
            def _():
                pltpu.make_async_copy(
                    rows_y, acc_sh.at[ridx_y], ssem_y).wait()

            @pl.when(k + 1 < CPT)
            def _():
                if x == 2:
                    @pl.when(lax.rem(k + 1, BLK) == 0)
                    def _():
                        load_block(k0 + k + 1)

                copy_idx(ridx_y, gidx_y, val_y, lax.rem(k + 1, BLK))
                fire_gather(gidx_y, rows_y, gsem_y)

            pltpu.make_async_copy(z_hbm.at[gidx_x], rows_x, gsem_x).wait()
            scale(rows_x, val_x)
            fire_scatter(rows_x, ridx_x, ssem_x)

        def tri_body(it, _):
            chunk_step(it, 0)
            chunk_step(it, 1)
            chunk_step(it, 2)
            return 0

        lax.fori_loop(0, NTRI, tri_body, 0)
        pltpu.make_async_copy(rows_b, acc_sh.at[ridx_b], ssem_b).wait()
        pltpu.make_async_copy(rows_c, acc_sh.at[ridx_c], ssem_c).wait()

        @pl.when(s < XTRA)
        def _():
            e0 = pl.multiple_of((NT * CPT + s) * CH, 8)
            pltpu.sync_copy(row_hbm.at[pl.ds(e0, CH)], brow.at[pl.ds(0, CH)])
            pltpu.sync_copy(col_hbm.at[pl.ds(e0, CH)], bcol.at[pl.ds(0, CH)])
            pltpu.sync_copy(val_hbm.at[pl.ds(e0, CH)], bval.at[pl.ds(0, CH)])
            copy_idx(ridx_a, gidx_a, val_a, 0)
            fire_gather(gidx_a, rows_a, gsem_a).wait()
            scale(rows_a, val_a)
            fire_scatter(rows_a, ridx_a, ssem_a).wait()

        plsc.subcore_barrier()

        eslots = (
            (rows_a.at[pl.ds(0, RCH)], rows_a,
             rows_b.at[pl.ds(0, RCH)], rows_b, wb0, ia0, ih0, ein0, eout0),
            (eacc1, eacc1, ez1, ez1, wb1, ia1, ih1, ein1, eout1),
        )

        def epi_offsets(ri):
            jc = s + NT * ri
            r0 = pl.multiple_of(jc * RCH, 8)
            g0 = pl.multiple_of(cM + r0, 8)
            return jc, r0, g0

        def epi_fire_in(q, ri):
            _, _, zd, _, wbq, _, ihq, einq, _ = eslots[q]
            jc, r0, g0 = epi_offsets(ri)

            @pl.when(jc < NRC)
            def _():
                set_idx(ihq, g0)
                pltpu.async_copy(z_hbm.at[ihq], zd, einq)
                if gamma != 0.0:
                    pltpu.async_copy(w_hbm.at[ihq], wbq, einq)

        eaccs = (rows_a, rows_c)

        def epi_wait_out(q, ri):
            accq = eaccs[q]
            ioq = eslots[q][5]
            jc, r0, g0 = epi_offsets(ri)

            @pl.when(jnp.logical_and(ri >= 0, jc < NRC))
            def _():
                pltpu.make_async_copy(
                    accq.at[pl.ds(0, RCH)], out_hbm.at[ioq],
                    eslots[q][8]).wait()

        def epi_process(q, ri):
            _, _, zd, zv, wbq, ioq, ihq, einq, eoutq = eslots[q]
            accq = eaccs[q]
            jc, r0, g0 = epi_offsets(ri)
            epi_wait_out(q, ri - 2)

            @pl.when(jc < NRC)
            def _():
                pltpu.sync_copy(acc_sh.at[pl.ds(r0, RCH)],
                                accq.at[pl.ds(0, RCH)])
                pltpu.make_async_copy(z_hbm.at[ihq], zd, einq).wait()
                if gamma != 0.0:
                    pltpu.make_async_copy(w_hbm.at[ihq], wbq, einq).wait()

                def comb_body(r, _):
                    for j in range(_FSL):
                        sl = pl.ds(_LANES * j, _LANES)
                        res = alpha * accq[r, sl] + beta * zv[r, sl]
                        if gamma != 0.0:
                            res = res + gamma * wbq[r, sl]
                        accq[r, sl] = res
                    return 0

                lax.fori_loop(0, RCH, comb_body, 0)
                set_idx(ioq, g0)
                pltpu.async_copy(accq.at[pl.ds(0, RCH)],
                                 out_hbm.at[ioq], eoutq)

        epi_fire_in(0, jnp.int32(0))
        epi_fire_in(1, jnp.int32(1))

        def epi_pair(t, _):
            ra = 2 * t
            rb = 2 * t + 1
            epi_process(0, ra)
            epi_fire_in(0, ra + 2)
            epi_process(1, rb)
            epi_fire_in(1, rb + 2)
            return 0

        lax.fori_loop(0, RR // 2, epi_pair, 0)
        epi_wait_out(0, jnp.int32(RR - 2))
        epi_wait_out(1, jnp.int32(RR - 1))

    return spmm


_spmm_first = _make_spmm(1.0, -1.0, 0.0)
_spmm_rec = _make_spmm(2.0, -2.0, -1.0)


TROW = 2000
NBLK = M // TROW


def _tail_body(z0_ref, z1_ref, z2_ref, z3_ref, w_ref, b_ref, out_ref):
    acc = jnp.dot(z0_ref[...], w_ref[0], preferred_element_type=jnp.float32)
    acc += jnp.dot(z1_ref[...], w_ref[1], preferred_element_type=jnp.float32)
    acc += jnp.dot(z2_ref[...], w_ref[2], preferred_element_type=jnp.float32)
    acc += jnp.dot(z3_ref[...], w_ref[3], preferred_element_type=jnp.float32)
    h = jnp.maximum(acc + b_ref[0:1, :], 0.0)
    h = h.reshape(TROW // 2, 2, FOUT).max(axis=1)
    out_ref[...] = h[None]


def _tail(z0, z1, z2, z3, wk, b2):
    zspec = pl.BlockSpec((TROW, FIN), lambda n, i: (n * NBLK + i, 0))
    return pl.pallas_call(
        _tail_body,
        grid=(NB, NBLK),
        in_specs=[
            zspec, zspec, zspec, zspec,
            pl.BlockSpec((KPOLY, FIN, FOUT), lambda n, i: (0, 0, 0)),
            pl.BlockSpec((8, FOUT), lambda n, i: (0, 0)),
        ],
        out_specs=pl.BlockSpec((1, TROW // 2, FOUT), lambda n, i: (n, i, 0)),
        out_shape=jax.ShapeDtypeStruct((NB, M // 2, FOUT), jnp.float32),
    )(z0, z1, z2, z3, wk, b2)


def kernel(x, edge_index, edge_vals, kernel, bias):
    row = edge_index[0]
    col = edge_index[1]
    z0 = x.reshape(NB * M, FIN)
    z1 = _spmm_first(z0, z0, row, col, edge_vals)
    z2 = _spmm_rec(z1, z0, row, col, edge_vals)
    z3 = _spmm_rec(z2, z1, row, col, edge_vals)
    wk = kernel.reshape(FIN, KPOLY, FOUT).transpose(1, 0, 2)
    b2 = jnp.broadcast_to(bias.reshape(1, FOUT), (8, FOUT))
    return _tail(z0, z1, z2, z3, wk, b2)
